# baseline (speedup 1.0000x reference)
import functools

import jax
import jax.numpy as jnp
from jax import lax
from jax.experimental import pallas as pl
from jax.experimental.pallas import tpu as pltpu
from jax.experimental.pallas import tpu_sc as plsc

_BT = 1024
_RT = 64
_KC = 2048


def _argmin_body(x_ref, e_ref, idx_ref):
    n_codes = e_ref.shape[0]
    x = x_ref[...]
    x_norm = jnp.sum(x * x, axis=1, keepdims=True)
    xm2 = x * (-2.0)
    e_all = e_ref[...]
    e_norm = jnp.sum(e_all * e_all, axis=1)
    lane = lax.broadcasted_iota(jnp.int32, (_RT, 128), 1)
    for r in range(_BT // _RT):
        xr = xm2[r * _RT:(r + 1) * _RT, :]
        xnr = x_norm[r * _RT:(r + 1) * _RT, :]
        m = jnp.full((_RT, 128), jnp.inf, jnp.float32)
        mi = jnp.zeros((_RT, 128), jnp.int32)
        for c in range(n_codes // _KC):
            xe2 = lax.dot_general(
                xr, e_all[c * _KC:(c + 1) * _KC, :],
                (((1,), (1,)), ((), ())),
                preferred_element_type=jnp.float32)
            for g in range(_KC // 128):
                base = c * _KC + g * 128
                en_g = e_norm[base:base + 128]
                dg = (xnr + en_g[None, :]) + xe2[:, g * 128:(g + 1) * 128]
                upd = dg < m
                m = jnp.where(upd, dg, m)
                mi = jnp.where(upd, lane + base, mi)
        gm = jnp.min(m, axis=1, keepdims=True)
        idx_ref[pl.ds(r * _RT, _RT)] = jnp.min(
            jnp.where(m == gm, mi, n_codes), axis=1)


def _nearest_indices(flat_x, embed):
    tokens, dim = flat_x.shape
    n_codes = embed.shape[0]
    return pl.pallas_call(
        _argmin_body,
        grid=(tokens // _BT,),
        in_specs=[
            pl.BlockSpec((_BT, dim), lambda i: (i, 0)),
            pl.BlockSpec((n_codes, dim), lambda i: (0, 0)),
        ],
        out_specs=pl.BlockSpec((_BT,), lambda i: (i,)),
        out_shape=jax.ShapeDtypeStruct((tokens,), jnp.int32),
    )(flat_x, embed)


@functools.cache
def _make_sc_gather(tokens, dim):
    info = plsc.get_sparse_core_info()
    nw = info.num_cores * info.num_subcores
    b_per_w = tokens // nw
    mesh = plsc.VectorSubcoreMesh(core_axis_name="c", subcore_axis_name="s")

    @functools.partial(
        pl.kernel, mesh=mesh,
        compiler_params=pltpu.CompilerParams(use_tc_tiling_on_sc=False),
        out_type=jax.ShapeDtypeStruct((tokens, dim), jnp.float32),
        scratch_types=[
            pltpu.VMEM((b_per_w,), jnp.int32),
            pltpu.VMEM((b_per_w, dim), jnp.float32),
            pltpu.SemaphoreType.DMA,
        ],
    )
    def gather(table_hbm, idx_hbm, out_hbm, idx_v, rows_v, sem):
        wid = lax.axis_index("s") * info.num_cores + lax.axis_index("c")
        base = wid * b_per_w
        pltpu.sync_copy(idx_hbm.at[pl.ds(base, b_per_w)], idx_v)
        pltpu.async_copy(table_hbm.at[idx_v], rows_v, sem).wait()
        pltpu.sync_copy(rows_v, out_hbm.at[pl.ds(base, b_per_w)])

    return gather


def kernel(x, embed):
    tokens = x.shape[0] * x.shape[1]
    dim = x.shape[2]
    flat_x = x.reshape(tokens, dim)
    idx = _nearest_indices(flat_x, embed)
    quantized = _make_sc_gather(tokens, dim)(embed, idx)
    return quantized.reshape(x.shape)


# speedup vs baseline: 1.3815x; 1.3815x over previous
import functools

import jax
import jax.numpy as jnp
from jax import lax
from jax.experimental import pallas as pl
from jax.experimental.pallas import tpu as pltpu
from jax.experimental.pallas import tpu_sc as plsc

_BT = 1024
_RT = 128
_KC = 2048


def _argmin_body(x_ref, e_ref, idx_ref, en_ref):
    n_codes = e_ref.shape[0]
    e_all = e_ref[...]

    @pl.when(pl.program_id(0) == 0)
    def _():
        en_ref[...] = jnp.sum(e_all * e_all, axis=1)

    x = x_ref[...]
    x_norm = jnp.sum(x * x, axis=1, keepdims=True)
    xm2 = x * (-2.0)
    e_norm = en_ref[...]
    lane = lax.broadcasted_iota(jnp.int32, (_RT, 128), 1)
    for r in range(_BT // _RT):
        xr = xm2[r * _RT:(r + 1) * _RT, :]
        xnr = x_norm[r * _RT:(r + 1) * _RT, :]
        m = jnp.full((_RT, 128), jnp.inf, jnp.float32)
        mi = jnp.zeros((_RT, 128), jnp.int32)
        for c in range(n_codes // _KC):
            xe2 = lax.dot_general(
                xr, e_all[c * _KC:(c + 1) * _KC, :],
                (((1,), (1,)), ((), ())),
                preferred_element_type=jnp.float32)
            for g in range(_KC // 128):
                base = c * _KC + g * 128
                en_g = e_norm[base:base + 128]
                dg = (xnr + en_g[None, :]) + xe2[:, g * 128:(g + 1) * 128]
                upd = dg < m
                m = jnp.where(upd, dg, m)
                mi = jnp.where(upd, lane + base, mi)
        gm = jnp.min(m, axis=1, keepdims=True)
        idx_ref[pl.ds(r * _RT, _RT)] = jnp.min(
            jnp.where(m == gm, mi, n_codes), axis=1)


def _nearest_indices(flat_x, embed):
    tokens, dim = flat_x.shape
    n_codes = embed.shape[0]
    return pl.pallas_call(
        _argmin_body,
        grid=(tokens // _BT,),
        in_specs=[
            pl.BlockSpec((_BT, dim), lambda i: (i, 0)),
            pl.BlockSpec((n_codes, dim), lambda i: (0, 0)),
        ],
        out_specs=pl.BlockSpec((_BT,), lambda i: (i,)),
        out_shape=jax.ShapeDtypeStruct((tokens,), jnp.int32),
        scratch_shapes=[pltpu.VMEM((n_codes,), jnp.float32)],
    )(flat_x, embed)


@functools.cache
def _make_sc_gather(tokens, dim):
    info = plsc.get_sparse_core_info()
    nw = info.num_cores * info.num_subcores
    b_per_w = tokens // nw
    mesh = plsc.VectorSubcoreMesh(core_axis_name="c", subcore_axis_name="s")

    @functools.partial(
        pl.kernel, mesh=mesh,
        compiler_params=pltpu.CompilerParams(use_tc_tiling_on_sc=False),
        out_type=jax.ShapeDtypeStruct((tokens, dim), jnp.float32),
        scratch_types=[
            pltpu.VMEM((b_per_w,), jnp.int32),
            pltpu.VMEM((b_per_w, dim), jnp.float32),
            pltpu.SemaphoreType.DMA,
        ],
    )
    def gather(table_hbm, idx_hbm, out_hbm, idx_v, rows_v, sem):
        wid = lax.axis_index("s") * info.num_cores + lax.axis_index("c")
        base = wid * b_per_w
        pltpu.sync_copy(idx_hbm.at[pl.ds(base, b_per_w)], idx_v)
        pltpu.async_copy(table_hbm.at[idx_v], rows_v, sem).wait()
        pltpu.sync_copy(rows_v, out_hbm.at[pl.ds(base, b_per_w)])

    return gather


def kernel(x, embed):
    tokens = x.shape[0] * x.shape[1]
    dim = x.shape[2]
    flat_x = x.reshape(tokens, dim)
    idx = _nearest_indices(flat_x, embed)
    quantized = _make_sc_gather(tokens, dim)(embed, idx)
    return quantized.reshape(x.shape)


# speedup vs baseline: 1.3825x; 1.0007x over previous
import functools

import jax
import jax.numpy as jnp
from jax import lax
from jax.experimental import pallas as pl
from jax.experimental.pallas import tpu as pltpu
from jax.experimental.pallas import tpu_sc as plsc

_BT = 1024
_RT = 128
_KC = 1024


def _argmin_body(x_ref, e_ref, idx_ref, en_ref):
    n_codes = e_ref.shape[0]
    e_all = e_ref[...]

    @pl.when(pl.program_id(0) == 0)
    def _():
        en_ref[...] = jnp.sum(e_all * e_all, axis=1)

    x = x_ref[...]
    x_norm = jnp.sum(x * x, axis=1, keepdims=True)
    xm2 = x * (-2.0)
    e_norm = en_ref[...]
    lane = lax.broadcasted_iota(jnp.int32, (_RT, 128), 1)
    for r in range(_BT // _RT):
        xr = xm2[r * _RT:(r + 1) * _RT, :]
        xnr = x_norm[r * _RT:(r + 1) * _RT, :]
        m = jnp.full((_RT, 128), jnp.inf, jnp.float32)
        mi = jnp.zeros((_RT, 128), jnp.int32)
        for c in range(n_codes // _KC):
            xe2 = lax.dot_general(
                xr, e_all[c * _KC:(c + 1) * _KC, :],
                (((1,), (1,)), ((), ())),
                preferred_element_type=jnp.float32)
            for g in range(_KC // 128):
                base = c * _KC + g * 128
                en_g = e_norm[base:base + 128]
                dg = (xnr + en_g[None, :]) + xe2[:, g * 128:(g + 1) * 128]
                upd = dg < m
                m = jnp.where(upd, dg, m)
                mi = jnp.where(upd, lane + base, mi)
        gm = jnp.min(m, axis=1, keepdims=True)
        idx_ref[pl.ds(r * _RT, _RT)] = jnp.min(
            jnp.where(m == gm, mi, n_codes), axis=1)


def _nearest_indices(flat_x, embed):
    tokens, dim = flat_x.shape
    n_codes = embed.shape[0]
    return pl.pallas_call(
        _argmin_body,
        grid=(tokens // _BT,),
        in_specs=[
            pl.BlockSpec((_BT, dim), lambda i: (i, 0)),
            pl.BlockSpec((n_codes, dim), lambda i: (0, 0)),
        ],
        out_specs=pl.BlockSpec((_BT,), lambda i: (i,)),
        out_shape=jax.ShapeDtypeStruct((tokens,), jnp.int32),
        scratch_shapes=[pltpu.VMEM((n_codes,), jnp.float32)],
    )(flat_x, embed)


@functools.cache
def _make_sc_gather(tokens, dim):
    info = plsc.get_sparse_core_info()
    nw = info.num_cores * info.num_subcores
    b_per_w = tokens // nw
    mesh = plsc.VectorSubcoreMesh(core_axis_name="c", subcore_axis_name="s")

    @functools.partial(
        pl.kernel, mesh=mesh,
        compiler_params=pltpu.CompilerParams(use_tc_tiling_on_sc=False),
        out_type=jax.ShapeDtypeStruct((tokens, dim), jnp.float32),
        scratch_types=[
            pltpu.VMEM((b_per_w,), jnp.int32),
            pltpu.VMEM((b_per_w, dim), jnp.float32),
            pltpu.SemaphoreType.DMA,
        ],
    )
    def gather(table_hbm, idx_hbm, out_hbm, idx_v, rows_v, sem):
        wid = lax.axis_index("s") * info.num_cores + lax.axis_index("c")
        base = wid * b_per_w
        pltpu.sync_copy(idx_hbm.at[pl.ds(base, b_per_w)], idx_v)
        pltpu.async_copy(table_hbm.at[idx_v], rows_v, sem).wait()
        pltpu.sync_copy(rows_v, out_hbm.at[pl.ds(base, b_per_w)])

    return gather


def kernel(x, embed):
    tokens = x.shape[0] * x.shape[1]
    dim = x.shape[2]
    flat_x = x.reshape(tokens, dim)
    idx = _nearest_indices(flat_x, embed)
    quantized = _make_sc_gather(tokens, dim)(embed, idx)
    return quantized.reshape(x.shape)
